# baseline (speedup 1.0000x reference)
import jax
import jax.numpy as jnp
from jax import lax


def _splat_lane(v, i):
    idx = jnp.full((16, 1), i, jnp.int32)
    dnums = lax.GatherDimensionNumbers(
        offset_dims=(), collapsed_slice_dims=(0,), start_index_map=(0,))
    return lax.gather(v, idx, dnums, (1,),
                      mode=lax.GatherScatterMode.PROMISE_IN_BOUNDS)
from jax.experimental import pallas as pl
from jax.experimental.pallas import tpu as pltpu
from jax.experimental.pallas import tpu_sc as plsc

N_DATA = 50000
N_TRUNC = 12500
ND_PAD = 51200
NT_PAD = 12800
AH_UP = ND_PAD // 2
E = 800000
F = 44
W = 32
NC = 2
NS = 16
LANES = 16

EPR = 128
WR = 4
WIN = EPR * WR
ROWS = 6272
E_PAD = ROWS * EPR
ROWS_PT = ROWS // NS

ZROWS = 256


def _body(xpad, dsrc, ddst, dwt, usrc, udst, uwt, cfg, xtr, out,
          acc_dn, acc_up, idx_v, dst_v, w_v, dstx_v, rows_v, zbuf, cfg_v,
          sem_e, sem_g, sem_a, sem_z):
    c = lax.axis_index("c")
    s = lax.axis_index("s")

    zvec = jnp.zeros((LANES,), jnp.float32)

    @pl.loop(0, ZROWS)
    def _(rr):
        zbuf[rr, pl.ds(0, LANES)] = zvec
        zbuf[rr, pl.ds(LANES, LANES)] = zvec

    pltpu.sync_copy(cfg, cfg_v)
    cfgv = cfg_v[pl.ds(0, LANES)]
    r1e = cfgv[0]
    r2s = cfgv[1]

    def spans(nrows, nz):
        q = nrows // NS
        lo = s * q
        return [jnp.minimum(lo + i * ZROWS, lo + q - ZROWS) for i in range(nz)]

    def zero_acc(acc, nrows, nz):
        sps = spans(nrows, nz)
        for st in sps:
            pltpu.async_copy(zbuf, acc.at[pl.ds(st, ZROWS)], sem_z)
        for st in sps:
            pltpu.make_async_copy(zbuf, acc.at[pl.ds(st, ZROWS)],
                                  sem_z).wait()

    def copy_out(acc, nrows, nz, dst_hbm, dst_base):
        sps = spans(nrows, nz)
        for st in sps:
            pltpu.async_copy(acc.at[pl.ds(st, ZROWS)],
                             dst_hbm.at[pl.ds(dst_base + st, ZROWS)], sem_z)
        for st in sps:
            pltpu.make_async_copy(acc.at[pl.ds(st, ZROWS)],
                                  dst_hbm.at[pl.ds(dst_base + st, ZROWS)],
                                  sem_z).wait()

    def stage(src_r, dst_r, wt_r, table, idx_mul, tbl_base, acc, acc_rows,
              nz, out_hbm, out_base, dlo, dhi, row_lo, row_hi):
        zero_acc(acc, acc_rows, nz)
        plsc.subcore_barrier()

        offv = jnp.broadcast_to(tbl_base, (LANES,)).astype(jnp.int32)
        dlov = jnp.broadcast_to(dlo, (LANES,)).astype(jnp.int32)
        dhiv = jnp.broadcast_to(dhi, (LANES,)).astype(jnp.int32)

        nrows = row_hi - row_lo
        pt = ((nrows + NS * WR - 1) // (NS * WR)) * WR
        t_lo = row_lo + jnp.minimum(s * pt, nrows)
        t_hi = row_lo + jnp.minimum((s + 1) * pt, nrows)
        nw = (t_hi - t_lo) // WR

        def fire_l(w, p):
            wrow = t_lo + w * WR
            pltpu.async_copy(src_r.at[pl.ds(wrow, WR)], idx_v.at[p], sem_e)
            pltpu.async_copy(dst_r.at[pl.ds(wrow, WR)], dst_v.at[p], sem_e)
            pltpu.async_copy(wt_r.at[pl.ds(wrow, WR)], w_v.at[p], sem_e)

        def wait_l(p):
            pltpu.make_async_copy(src_r.at[pl.ds(0, WR)], idx_v.at[p],
                                  sem_e).wait()
            pltpu.make_async_copy(dst_r.at[pl.ds(0, WR)], dst_v.at[p],
                                  sem_e).wait()
            pltpu.make_async_copy(wt_r.at[pl.ds(0, WR)], w_v.at[p],
                                  sem_e).wait()

        def prep(p):
            @pl.loop(0, WR)
            def _(r):
                for k in range(EPR // LANES):
                    sl = pl.ds(k * LANES, LANES)
                    idx_v[p, r, sl] = idx_v[p, r, sl] * idx_mul + offv
                    d = dst_v[p, r, sl]
                    inr = jnp.logical_and(d >= dlov, d < dhiv)
                    w_v[p, r, sl] = jnp.where(inr, w_v[p, r, sl], 0.0)
                    dstx_v[p, r, sl] = jnp.clip(d - dlov, 0, acc_rows - 1)

        def fire_g(p):
            for r in range(WR):
                pltpu.async_copy(table.at[idx_v.at[p, r]],
                                 rows_v.at[p, pl.ds(r * EPR, EPR)], sem_g)

        def wait_g(p):
            for r in range(WR):
                pltpu.make_async_copy(table.at[idx_v.at[p, r]],
                                      rows_v.at[p, pl.ds(r * EPR, EPR)],
                                      sem_g).wait()

        def scale(p):
            IL = 4

            @pl.loop(0, WR)
            def _(r):
                for k in range(EPR // LANES):
                    wv = w_v[p, r, pl.ds(k * LANES, LANES)]
                    for i0 in range(0, LANES, IL):
                        es = [r * EPR + k * LANES + i0 + j for j in range(IL)]
                        wss = [_splat_lane(wv, i0 + j) for j in range(IL)]
                        vals = []
                        for e in es:
                            vals.append(rows_v[p, e, pl.ds(0, LANES)])
                            vals.append(rows_v[p, e, pl.ds(LANES, LANES)])
                        outs = [v * wss[j // 2] for j, v in enumerate(vals)]
                        for j, e in enumerate(es):
                            rows_v[p, e, pl.ds(0, LANES)] = outs[2 * j]
                            rows_v[p, e, pl.ds(LANES, LANES)] = outs[2 * j + 1]

        def fire_a(p):
            for r in range(WR):
                pltpu.async_copy(rows_v.at[p, pl.ds(r * EPR, EPR)],
                                 acc.at[dstx_v.at[p, r]], sem_a, add=True)

        def wait_a(p):
            for r in range(WR):
                pltpu.make_async_copy(rows_v.at[p, pl.ds(r * EPR, EPR)],
                                      acc.at[dstx_v.at[p, r]], sem_a).wait()

        @pl.when(nw > 0)
        def _():
            fire_l(0, 0)

            @pl.loop(0, nw)
            def _(w):
                p = lax.rem(w, 2)
                wait_l(p)

                @pl.when(w >= 2)
                def _():
                    wait_a(p)

                prep(p)
                fire_g(p)

                @pl.when(w >= 1)
                def _():
                    wait_g(1 - p)
                    scale(1 - p)
                    fire_a(1 - p)

                @pl.when(w + 1 < nw)
                def _():
                    fire_l(w + 1, 1 - p)

            pl_ = lax.rem(nw - 1, 2)
            wait_g(pl_)
            scale(pl_)
            fire_a(pl_)
            wait_a(pl_)

            @pl.when(nw > 1)
            def _():
                wait_a(1 - pl_)

        plsc.subcore_barrier()
        copy_out(acc, acc_rows, nz, out_hbm, out_base)
        plsc.subcore_barrier()

    @pl.loop(0, 2)
    def _(b):
        bh = b * NC + c
        stage(dsrc, ddst, dwt, xpad, NC, b * (N_DATA * NC) + c, acc_dn,
              NT_PAD, 4, xtr, bh * NT_PAD, 0, NT_PAD, 0, ROWS)

        row_bounds = ((0, r1e), (r2s, ROWS))
        for h in range(2):
            dlo = h * AH_UP
            rl, rh = row_bounds[h]
            stage(usrc, udst, uwt, xtr, 1, bh * NT_PAD, acc_up, AH_UP, 7,
                  out, bh * ND_PAD + dlo, dlo, dlo + AH_UP, rl, rh)


@jax.jit
def _run(xpad, dsrc, ddst, dwt, usrc, udst, uwt, cfg):
    mesh = plsc.VectorSubcoreMesh(core_axis_name="c", subcore_axis_name="s")
    f = pl.kernel(
        _body,
        out_type=(
            jax.ShapeDtypeStruct((2 * NC * NT_PAD, W), jnp.float32),
            jax.ShapeDtypeStruct((2 * NC * ND_PAD, W), jnp.float32),
        ),
        mesh=mesh,
        compiler_params=pltpu.CompilerParams(use_tc_tiling_on_sc=False,
                                             disable_bounds_checks=True),
        scratch_types=[
            pltpu.VMEM_SHARED((NT_PAD, W), jnp.float32),
            pltpu.VMEM_SHARED((AH_UP, W), jnp.float32),
            pltpu.VMEM((2, WR, EPR), jnp.int32),
            pltpu.VMEM((2, WR, EPR), jnp.int32),
            pltpu.VMEM((2, WR, EPR), jnp.float32),
            pltpu.VMEM((2, WR, EPR), jnp.int32),
            pltpu.VMEM((2, WIN, W), jnp.float32),
            pltpu.VMEM((ZROWS, W), jnp.float32),
            pltpu.VMEM((LANES,), jnp.int32),
            pltpu.SemaphoreType.DMA,
            pltpu.SemaphoreType.DMA,
            pltpu.SemaphoreType.DMA,
            pltpu.SemaphoreType.DMA,
        ],
    )
    return f(xpad, dsrc, ddst, dwt, usrc, udst, uwt, cfg)


def _pad_edges(a, fill=0):
    return jnp.concatenate(
        [a, jnp.full((E_PAD - E,), fill, a.dtype)]).reshape(ROWS, EPR)


def kernel(x, down_src, down_dst, down_weight, up_src, up_dst, up_weight):
    b, t, en, n, f = x.shape
    x2 = x.reshape(b * t * en, n, f)
    xpad = jnp.pad(x2, ((0, 0), (0, 0), (0, 2 * W - f)))
    xpad = xpad.reshape(b * t * en * n * NC, W)

    m0 = jnp.searchsorted(up_dst, AH_UP).astype(jnp.int32)
    r1e = ((m0 + WIN - 1) // WIN) * WR
    r2s = (m0 // WIN) * WR
    cfg = jnp.zeros((LANES,), jnp.int32).at[0].set(r1e).at[1].set(r2s)

    _, outp = _run(
        xpad,
        _pad_edges(down_src), _pad_edges(down_dst, N_TRUNC - 1),
        _pad_edges(down_weight),
        _pad_edges(up_src), _pad_edges(up_dst, N_DATA - 1),
        _pad_edges(up_weight),
        cfg,
    )
    outp = outp.reshape(b * t * en, NC, ND_PAD, W)[:, :, :n]
    outp = outp.transpose(0, 2, 1, 3).reshape(b * t * en, n, NC * W)[:, :, :f]
    return outp.reshape(b, t, en, n, f)


# speedup vs baseline: 1.0406x; 1.0406x over previous
import jax
import jax.numpy as jnp
from jax import lax


def _splat_lane(v, i):
    idx = jnp.full((16, 1), i, jnp.int32)
    dnums = lax.GatherDimensionNumbers(
        offset_dims=(), collapsed_slice_dims=(0,), start_index_map=(0,))
    return lax.gather(v, idx, dnums, (1,),
                      mode=lax.GatherScatterMode.PROMISE_IN_BOUNDS)
from jax.experimental import pallas as pl
from jax.experimental.pallas import tpu as pltpu
from jax.experimental.pallas import tpu_sc as plsc

N_DATA = 50000
N_TRUNC = 12500
ND_PAD = 51200
NT_PAD = 12800
AH_UP = ND_PAD // 2
E = 800000
F = 44
W = 32
NC = 2
NS = 16
LANES = 16

EPR = 128
WR = 4
WIN = EPR * WR
ROWS = 6272
E_PAD = ROWS * EPR
ROWS_PT = ROWS // NS

ZROWS = 256


def _body(xpad, dsrc, ddst, dwt, usrc, udst, uwt, cfg, xtr, out,
          acc_dn, acc_up, idx_v, dst_v, w_v, dstx_v, rows_v, zbuf, cfg_v,
          sem_e, sem_g, sem_a, sem_z):
    c = lax.axis_index("c")
    s = lax.axis_index("s")

    zvec = jnp.zeros((LANES,), jnp.float32)

    @pl.loop(0, ZROWS)
    def _(rr):
        zbuf[rr, pl.ds(0, LANES)] = zvec
        zbuf[rr, pl.ds(LANES, LANES)] = zvec

    pltpu.sync_copy(cfg, cfg_v)
    cfgv = cfg_v[pl.ds(0, LANES)]
    r1e = cfgv[0]
    r2s = cfgv[1]

    def spans(nrows, nz):
        q = nrows // NS
        lo = s * q
        return [jnp.minimum(lo + i * ZROWS, lo + q - ZROWS) for i in range(nz)]

    def zero_acc(acc, nrows, nz):
        sps = spans(nrows, nz)
        for st in sps:
            pltpu.async_copy(zbuf, acc.at[pl.ds(st, ZROWS)], sem_z)
        for st in sps:
            pltpu.make_async_copy(zbuf, acc.at[pl.ds(st, ZROWS)],
                                  sem_z).wait()

    def copy_out(acc, nrows, nz, dst_hbm, dst_base):
        sps = spans(nrows, nz)
        for st in sps:
            pltpu.async_copy(acc.at[pl.ds(st, ZROWS)],
                             dst_hbm.at[pl.ds(dst_base + st, ZROWS)], sem_z)
        for st in sps:
            pltpu.make_async_copy(acc.at[pl.ds(st, ZROWS)],
                                  dst_hbm.at[pl.ds(dst_base + st, ZROWS)],
                                  sem_z).wait()

    def stage(src_r, dst_r, wt_r, table, idx_mul, tbl_base, acc, acc_rows,
              nz, out_hbm, out_base, dlo, dhi, row_lo, row_hi):
        zero_acc(acc, acc_rows, nz)
        plsc.subcore_barrier()

        offv = jnp.broadcast_to(tbl_base, (LANES,)).astype(jnp.int32)
        dlov = jnp.broadcast_to(dlo, (LANES,)).astype(jnp.int32)
        dhiv = jnp.broadcast_to(dhi, (LANES,)).astype(jnp.int32)

        nrows = row_hi - row_lo
        pt = ((nrows + NS * WR - 1) // (NS * WR)) * WR
        t_lo = row_lo + jnp.minimum(s * pt, nrows)
        t_hi = row_lo + jnp.minimum((s + 1) * pt, nrows)
        nw = (t_hi - t_lo) // WR

        def fire_l(w, p):
            wrow = t_lo + w * WR
            pltpu.async_copy(src_r.at[pl.ds(wrow, WR)], idx_v.at[p], sem_e)
            pltpu.async_copy(dst_r.at[pl.ds(wrow, WR)], dst_v.at[p], sem_e)
            pltpu.async_copy(wt_r.at[pl.ds(wrow, WR)], w_v.at[p], sem_e)

        def wait_l(p):
            pltpu.make_async_copy(src_r.at[pl.ds(0, WR)], idx_v.at[p],
                                  sem_e).wait()
            pltpu.make_async_copy(dst_r.at[pl.ds(0, WR)], dst_v.at[p],
                                  sem_e).wait()
            pltpu.make_async_copy(wt_r.at[pl.ds(0, WR)], w_v.at[p],
                                  sem_e).wait()

        def prep(p):
            ILP = 4

            @pl.loop(0, WR)
            def _(r):
                for k0 in range(0, EPR // LANES, ILP):
                    sls = [pl.ds((k0 + j) * LANES, LANES) for j in range(ILP)]
                    ix = [idx_v[p, r, sl] for sl in sls]
                    ds_ = [dst_v[p, r, sl] for sl in sls]
                    ws_ = [w_v[p, r, sl] for sl in sls]
                    ixo = [v * idx_mul + offv for v in ix]
                    inr = [jnp.logical_and(d >= dlov, d < dhiv) for d in ds_]
                    wo = [jnp.where(m, w, 0.0) for m, w in zip(inr, ws_)]
                    dxo = [jnp.clip(d - dlov, 0, acc_rows - 1) for d in ds_]
                    for j, sl in enumerate(sls):
                        idx_v[p, r, sl] = ixo[j]
                        w_v[p, r, sl] = wo[j]
                        dstx_v[p, r, sl] = dxo[j]

        def fire_g(p):
            for r in range(WR):
                pltpu.async_copy(table.at[idx_v.at[p, r]],
                                 rows_v.at[p, pl.ds(r * EPR, EPR)], sem_g)

        def wait_g(p):
            for r in range(WR):
                pltpu.make_async_copy(table.at[idx_v.at[p, r]],
                                      rows_v.at[p, pl.ds(r * EPR, EPR)],
                                      sem_g).wait()

        def scale(p):
            IL = 4

            @pl.loop(0, WR)
            def _(r):
                for k in range(EPR // LANES):
                    wv = w_v[p, r, pl.ds(k * LANES, LANES)]
                    for i0 in range(0, LANES, IL):
                        es = [r * EPR + k * LANES + i0 + j for j in range(IL)]
                        wss = [_splat_lane(wv, i0 + j) for j in range(IL)]
                        vals = []
                        for e in es:
                            vals.append(rows_v[p, e, pl.ds(0, LANES)])
                            vals.append(rows_v[p, e, pl.ds(LANES, LANES)])
                        outs = [v * wss[j // 2] for j, v in enumerate(vals)]
                        for j, e in enumerate(es):
                            rows_v[p, e, pl.ds(0, LANES)] = outs[2 * j]
                            rows_v[p, e, pl.ds(LANES, LANES)] = outs[2 * j + 1]

        def fire_a(p):
            for r in range(WR):
                pltpu.async_copy(rows_v.at[p, pl.ds(r * EPR, EPR)],
                                 acc.at[dstx_v.at[p, r]], sem_a, add=True)

        def wait_a(p):
            for r in range(WR):
                pltpu.make_async_copy(rows_v.at[p, pl.ds(r * EPR, EPR)],
                                      acc.at[dstx_v.at[p, r]], sem_a).wait()

        @pl.when(nw > 0)
        def _():
            fire_l(0, 0)

            @pl.loop(0, nw)
            def _(w):
                p = lax.rem(w, 2)
                wait_l(p)

                @pl.when(w >= 2)
                def _():
                    wait_a(p)

                prep(p)
                fire_g(p)

                @pl.when(w >= 1)
                def _():
                    wait_g(1 - p)
                    scale(1 - p)
                    fire_a(1 - p)

                @pl.when(w + 1 < nw)
                def _():
                    fire_l(w + 1, 1 - p)

            pl_ = lax.rem(nw - 1, 2)
            wait_g(pl_)
            scale(pl_)
            fire_a(pl_)
            wait_a(pl_)

            @pl.when(nw > 1)
            def _():
                wait_a(1 - pl_)

        plsc.subcore_barrier()
        copy_out(acc, acc_rows, nz, out_hbm, out_base)
        plsc.subcore_barrier()

    @pl.loop(0, 2)
    def _(b):
        bh = b * NC + c
        stage(dsrc, ddst, dwt, xpad, NC, b * (N_DATA * NC) + c, acc_dn,
              NT_PAD, 4, xtr, bh * NT_PAD, 0, NT_PAD, 0, ROWS)

        row_bounds = ((0, r1e), (r2s, ROWS))
        for h in range(2):
            dlo = h * AH_UP
            rl, rh = row_bounds[h]
            stage(usrc, udst, uwt, xtr, 1, bh * NT_PAD, acc_up, AH_UP, 7,
                  out, bh * ND_PAD + dlo, dlo, dlo + AH_UP, rl, rh)


@jax.jit
def _run(xpad, dsrc, ddst, dwt, usrc, udst, uwt, cfg):
    mesh = plsc.VectorSubcoreMesh(core_axis_name="c", subcore_axis_name="s")
    f = pl.kernel(
        _body,
        out_type=(
            jax.ShapeDtypeStruct((2 * NC * NT_PAD, W), jnp.float32),
            jax.ShapeDtypeStruct((2 * NC * ND_PAD, W), jnp.float32),
        ),
        mesh=mesh,
        compiler_params=pltpu.CompilerParams(use_tc_tiling_on_sc=False,
                                             disable_bounds_checks=True),
        scratch_types=[
            pltpu.VMEM_SHARED((NT_PAD, W), jnp.float32),
            pltpu.VMEM_SHARED((AH_UP, W), jnp.float32),
            pltpu.VMEM((2, WR, EPR), jnp.int32),
            pltpu.VMEM((2, WR, EPR), jnp.int32),
            pltpu.VMEM((2, WR, EPR), jnp.float32),
            pltpu.VMEM((2, WR, EPR), jnp.int32),
            pltpu.VMEM((2, WIN, W), jnp.float32),
            pltpu.VMEM((ZROWS, W), jnp.float32),
            pltpu.VMEM((LANES,), jnp.int32),
            pltpu.SemaphoreType.DMA,
            pltpu.SemaphoreType.DMA,
            pltpu.SemaphoreType.DMA,
            pltpu.SemaphoreType.DMA,
        ],
    )
    return f(xpad, dsrc, ddst, dwt, usrc, udst, uwt, cfg)


def _pad_edges(a, fill=0):
    return jnp.concatenate(
        [a, jnp.full((E_PAD - E,), fill, a.dtype)]).reshape(ROWS, EPR)


def kernel(x, down_src, down_dst, down_weight, up_src, up_dst, up_weight):
    b, t, en, n, f = x.shape
    x2 = x.reshape(b * t * en, n, f)
    xpad = jnp.pad(x2, ((0, 0), (0, 0), (0, 2 * W - f)))
    xpad = xpad.reshape(b * t * en * n * NC, W)

    m0 = jnp.searchsorted(up_dst, AH_UP).astype(jnp.int32)
    r1e = ((m0 + WIN - 1) // WIN) * WR
    r2s = (m0 // WIN) * WR
    cfg = jnp.zeros((LANES,), jnp.int32).at[0].set(r1e).at[1].set(r2s)

    _, outp = _run(
        xpad,
        _pad_edges(down_src), _pad_edges(down_dst, N_TRUNC - 1),
        _pad_edges(down_weight),
        _pad_edges(up_src), _pad_edges(up_dst, N_DATA - 1),
        _pad_edges(up_weight),
        cfg,
    )
    outp = outp.reshape(b * t * en, NC, ND_PAD, W)[:, :, :n]
    outp = outp.transpose(0, 2, 1, 3).reshape(b * t * en, n, NC * W)[:, :, :f]
    return outp.reshape(b, t, en, n, f)


# speedup vs baseline: 1.1763x; 1.1305x over previous
import jax
import jax.numpy as jnp
from jax import lax


def _splat_lane(v, i):
    idx = jnp.full((16, 1), i, jnp.int32)
    dnums = lax.GatherDimensionNumbers(
        offset_dims=(), collapsed_slice_dims=(0,), start_index_map=(0,))
    return lax.gather(v, idx, dnums, (1,),
                      mode=lax.GatherScatterMode.PROMISE_IN_BOUNDS)
from jax.experimental import pallas as pl
from jax.experimental.pallas import tpu as pltpu
from jax.experimental.pallas import tpu_sc as plsc

N_DATA = 50000
N_TRUNC = 12500
ND_PAD = 51200
NT_PAD = 12800
AH_UP = ND_PAD // 2
E = 800000
F = 44
W = 32
NC = 2
NS = 16
LANES = 16

EPR = 128
WR = 4
WIN = EPR * WR
ROWS = 6272
E_PAD = ROWS * EPR
ROWS_PT = ROWS // NS

ZROWS = 256


def _body(xpad, dsrc, ddst, dwt, usrc, udst, uwt, cfg, xtr, out,
          acc_dn, acc_up, idx_v, dst_v, w_v, dstx_v, rows_v, zbuf, cfg_v,
          sem_e, sem_g, sem_a, sem_z):
    c = lax.axis_index("c")
    s = lax.axis_index("s")

    zvec = jnp.zeros((LANES,), jnp.float32)

    @pl.loop(0, ZROWS)
    def _(rr):
        zbuf[rr, pl.ds(0, LANES)] = zvec
        zbuf[rr, pl.ds(LANES, LANES)] = zvec

    pltpu.sync_copy(cfg, cfg_v)
    cfgv = cfg_v[pl.ds(0, LANES)]
    r1e = cfgv[0]
    r2s = cfgv[1]

    def spans(nrows, nz):
        q = nrows // NS
        lo = s * q
        return [jnp.minimum(lo + i * ZROWS, lo + q - ZROWS) for i in range(nz)]

    def zero_acc(acc, nrows, nz):
        sps = spans(nrows, nz)
        for st in sps:
            pltpu.async_copy(zbuf, acc.at[pl.ds(st, ZROWS)], sem_z)
        for st in sps:
            pltpu.make_async_copy(zbuf, acc.at[pl.ds(st, ZROWS)],
                                  sem_z).wait()

    def copy_out(acc, nrows, nz, dst_hbm, dst_base):
        sps = spans(nrows, nz)
        for st in sps:
            pltpu.async_copy(acc.at[pl.ds(st, ZROWS)],
                             dst_hbm.at[pl.ds(dst_base + st, ZROWS)], sem_z)
        for st in sps:
            pltpu.make_async_copy(acc.at[pl.ds(st, ZROWS)],
                                  dst_hbm.at[pl.ds(dst_base + st, ZROWS)],
                                  sem_z).wait()

    def copy_out_final(acc, out_hbm, row0, max_rows):
        q = AH_UP // NS
        lo = s * q
        hi = jnp.minimum(lo + q, max_rows)
        for i in range(7):
            st = jnp.minimum(lo + i * ZROWS, hi - ZROWS)

            @pl.when(c == 0)
            def _():
                pltpu.sync_copy(
                    acc.at[pl.ds(st, ZROWS)],
                    out_hbm.at[pl.ds(row0 + st, ZROWS), pl.ds(0, W)])

            @pl.when(c == 1)
            def _():
                pltpu.sync_copy(
                    acc.at[pl.ds(st, ZROWS), pl.ds(0, 16)],
                    out_hbm.at[pl.ds(row0 + st, ZROWS), pl.ds(W, 16)])

    def stage(src_r, dst_r, wt_r, table, idx_mul, tbl_base, acc, acc_rows,
              nz, writeback, dlo, dhi, row_lo, row_hi):
        zero_acc(acc, acc_rows, nz)
        plsc.subcore_barrier()

        offv = jnp.broadcast_to(tbl_base, (LANES,)).astype(jnp.int32)
        dlov = jnp.broadcast_to(dlo, (LANES,)).astype(jnp.int32)
        dhiv = jnp.broadcast_to(dhi, (LANES,)).astype(jnp.int32)

        nrows = row_hi - row_lo
        pt = ((nrows + NS * WR - 1) // (NS * WR)) * WR
        t_lo = row_lo + jnp.minimum(s * pt, nrows)
        t_hi = row_lo + jnp.minimum((s + 1) * pt, nrows)
        nw = (t_hi - t_lo) // WR

        def fire_l(w, p):
            wrow = t_lo + w * WR
            pltpu.async_copy(src_r.at[pl.ds(wrow, WR)], idx_v.at[p], sem_e)
            pltpu.async_copy(dst_r.at[pl.ds(wrow, WR)], dst_v.at[p], sem_e)
            pltpu.async_copy(wt_r.at[pl.ds(wrow, WR)], w_v.at[p], sem_e)

        def wait_l(p):
            pltpu.make_async_copy(src_r.at[pl.ds(0, WR)], idx_v.at[p],
                                  sem_e).wait()
            pltpu.make_async_copy(dst_r.at[pl.ds(0, WR)], dst_v.at[p],
                                  sem_e).wait()
            pltpu.make_async_copy(wt_r.at[pl.ds(0, WR)], w_v.at[p],
                                  sem_e).wait()

        def prep(p):
            ILP = 4

            @pl.loop(0, WR)
            def _(r):
                for k0 in range(0, EPR // LANES, ILP):
                    sls = [pl.ds((k0 + j) * LANES, LANES) for j in range(ILP)]
                    ix = [idx_v[p, r, sl] for sl in sls]
                    ds_ = [dst_v[p, r, sl] for sl in sls]
                    ws_ = [w_v[p, r, sl] for sl in sls]
                    ixo = [v * idx_mul + offv for v in ix]
                    inr = [jnp.logical_and(d >= dlov, d < dhiv) for d in ds_]
                    wo = [jnp.where(m, w, 0.0) for m, w in zip(inr, ws_)]
                    dxo = [jnp.clip(d - dlov, 0, acc_rows - 1) for d in ds_]
                    for j, sl in enumerate(sls):
                        idx_v[p, r, sl] = ixo[j]
                        w_v[p, r, sl] = wo[j]
                        dstx_v[p, r, sl] = dxo[j]

        def fire_g(p):
            for r in range(WR):
                pltpu.async_copy(table.at[idx_v.at[p, r]],
                                 rows_v.at[p, pl.ds(r * EPR, EPR)], sem_g)

        def wait_g(p):
            for r in range(WR):
                pltpu.make_async_copy(table.at[idx_v.at[p, r]],
                                      rows_v.at[p, pl.ds(r * EPR, EPR)],
                                      sem_g).wait()

        def scale(p):
            IL = 4

            @pl.loop(0, WR)
            def _(r):
                for k in range(EPR // LANES):
                    wv = w_v[p, r, pl.ds(k * LANES, LANES)]
                    for i0 in range(0, LANES, IL):
                        es = [r * EPR + k * LANES + i0 + j for j in range(IL)]
                        wss = [_splat_lane(wv, i0 + j) for j in range(IL)]
                        vals = []
                        for e in es:
                            vals.append(rows_v[p, e, pl.ds(0, LANES)])
                            vals.append(rows_v[p, e, pl.ds(LANES, LANES)])
                        outs = [v * wss[j // 2] for j, v in enumerate(vals)]
                        for j, e in enumerate(es):
                            rows_v[p, e, pl.ds(0, LANES)] = outs[2 * j]
                            rows_v[p, e, pl.ds(LANES, LANES)] = outs[2 * j + 1]

        def fire_a(p):
            for r in range(WR):
                pltpu.async_copy(rows_v.at[p, pl.ds(r * EPR, EPR)],
                                 acc.at[dstx_v.at[p, r]], sem_a, add=True)

        def wait_a(p):
            for r in range(WR):
                pltpu.make_async_copy(rows_v.at[p, pl.ds(r * EPR, EPR)],
                                      acc.at[dstx_v.at[p, r]], sem_a).wait()

        @pl.when(nw > 0)
        def _():
            fire_l(0, 0)

            @pl.loop(0, nw)
            def _(w):
                p = lax.rem(w, 2)
                wait_l(p)

                @pl.when(w >= 2)
                def _():
                    wait_a(p)

                prep(p)
                fire_g(p)

                @pl.when(w >= 1)
                def _():
                    wait_g(1 - p)
                    scale(1 - p)
                    fire_a(1 - p)

                @pl.when(w + 1 < nw)
                def _():
                    fire_l(w + 1, 1 - p)

            pl_ = lax.rem(nw - 1, 2)
            wait_g(pl_)
            scale(pl_)
            fire_a(pl_)
            wait_a(pl_)

            @pl.when(nw > 1)
            def _():
                wait_a(1 - pl_)

        plsc.subcore_barrier()
        writeback()
        plsc.subcore_barrier()

    @pl.loop(0, 2)
    def _(b):
        bh = b * NC + c
        stage(dsrc, ddst, dwt, xpad, NC, b * (N_DATA * NC) + c, acc_dn,
              NT_PAD, 4,
              lambda: copy_out(acc_dn, NT_PAD, 4, xtr, bh * NT_PAD),
              0, NT_PAD, 0, ROWS)

        row_bounds = ((0, r1e), (r2s, ROWS))
        for h in range(2):
            dlo = h * AH_UP
            rl, rh = row_bounds[h]
            stage(usrc, udst, uwt, xtr, 1, bh * NT_PAD, acc_up, AH_UP, 7,
                  lambda: copy_out_final(acc_up, out, b * N_DATA + dlo,
                                         N_DATA - dlo),
                  dlo, dlo + AH_UP, rl, rh)


@jax.jit
def _run(xpad, dsrc, ddst, dwt, usrc, udst, uwt, cfg):
    mesh = plsc.VectorSubcoreMesh(core_axis_name="c", subcore_axis_name="s")
    f = pl.kernel(
        _body,
        out_type=(
            jax.ShapeDtypeStruct((2 * NC * NT_PAD, W), jnp.float32),
            jax.ShapeDtypeStruct((2 * N_DATA, W + 16), jnp.float32),
        ),
        mesh=mesh,
        compiler_params=pltpu.CompilerParams(use_tc_tiling_on_sc=False,
                                             disable_bounds_checks=True),
        scratch_types=[
            pltpu.VMEM_SHARED((NT_PAD, W), jnp.float32),
            pltpu.VMEM_SHARED((AH_UP, W), jnp.float32),
            pltpu.VMEM((2, WR, EPR), jnp.int32),
            pltpu.VMEM((2, WR, EPR), jnp.int32),
            pltpu.VMEM((2, WR, EPR), jnp.float32),
            pltpu.VMEM((2, WR, EPR), jnp.int32),
            pltpu.VMEM((2, WIN, W), jnp.float32),
            pltpu.VMEM((ZROWS, W), jnp.float32),
            pltpu.VMEM((LANES,), jnp.int32),
            pltpu.SemaphoreType.DMA,
            pltpu.SemaphoreType.DMA,
            pltpu.SemaphoreType.DMA,
            pltpu.SemaphoreType.DMA,
        ],
    )
    return f(xpad, dsrc, ddst, dwt, usrc, udst, uwt, cfg)


def _pad_edges(a, fill=0):
    return jnp.concatenate(
        [a, jnp.full((E_PAD - E,), fill, a.dtype)]).reshape(ROWS, EPR)


def kernel(x, down_src, down_dst, down_weight, up_src, up_dst, up_weight):
    b, t, en, n, f = x.shape
    x2 = x.reshape(b * t * en, n, f)
    xpad = jnp.pad(x2, ((0, 0), (0, 0), (0, 2 * W - f)))
    xpad = xpad.reshape(b * t * en * n * NC, W)

    m0 = jnp.searchsorted(up_dst, AH_UP).astype(jnp.int32)
    r1e = ((m0 + WIN - 1) // WIN) * WR
    r2s = (m0 // WIN) * WR
    cfg = jnp.zeros((LANES,), jnp.int32).at[0].set(r1e).at[1].set(r2s)

    _, outp = _run(
        xpad,
        _pad_edges(down_src), _pad_edges(down_dst, N_TRUNC - 1),
        _pad_edges(down_weight),
        _pad_edges(up_src), _pad_edges(up_dst, N_DATA - 1),
        _pad_edges(up_weight),
        cfg,
    )
    return outp[:, :f].reshape(b, t, en, n, f)
